# baseline (speedup 1.0000x reference)
import jax
import jax.numpy as jnp
from jax.experimental import pallas as pl

_S_BLK = 256


def _add_kernel(x_ref, emb_ref, o_ref):
    o_ref[...] = x_ref[...] + emb_ref[...]


def kernel(x, emb_weight):
    B, S, D = x.shape
    grid = (S // _S_BLK, B)
    return pl.pallas_call(
        _add_kernel,
        grid=grid,
        in_specs=[
            pl.BlockSpec((1, _S_BLK, D), lambda i, j: (j, i, 0)),
            pl.BlockSpec((_S_BLK, D), lambda i, j: (i, 0)),
        ],
        out_specs=pl.BlockSpec((1, _S_BLK, D), lambda i, j: (j, i, 0)),
        out_shape=jax.ShapeDtypeStruct((B, S, D), x.dtype),
    )(x, emb_weight)


# speedup vs baseline: 1.9626x; 1.9626x over previous
import jax
import jax.numpy as jnp
from jax.experimental import pallas as pl
from jax.experimental.pallas import tpu as pltpu


def _add_kernel(x_ref, emb_ref, o_ref):
    o_ref[...] = x_ref[...] + emb_ref[...]


def kernel(x, emb_weight):
    B, S, D = x.shape
    grid = (B,)
    return pl.pallas_call(
        _add_kernel,
        grid=grid,
        in_specs=[
            pl.BlockSpec((1, S, D), lambda j: (j, 0, 0)),
            pl.BlockSpec((S, D), lambda j: (0, 0)),
        ],
        out_specs=pl.BlockSpec((1, S, D), lambda j: (j, 0, 0)),
        out_shape=jax.ShapeDtypeStruct((B, S, D), x.dtype),
        compiler_params=pltpu.CompilerParams(
            dimension_semantics=("arbitrary",),
        ),
    )(x, emb_weight)
